# baseline (speedup 1.0000x reference)
import functools

import jax
import jax.numpy as jnp
from jax import lax
from jax.experimental import pallas as pl
from jax.experimental.pallas import tpu as pltpu
from jax.experimental.pallas import tpu_sc as plsc

N_ITEMS = 100001
D = 128
B = 4096
H = 200
TOTAL = B * H
SCALE = 50.0 ** 0.5

NC = 2
NS = 16
NW = NC * NS
PER_W = TOTAL // NW
CHUNK = 128
NCHUNK = PER_W // CHUNK
NGRP = NCHUNK // 2
LANES = 16


def _emb_body(x_hbm, table_hbm, out_hbm, idx_all,
              rows_g0, rows_g1, rows_s0, rows_s1,
              sg0, sg1, ss0, ss1):
    wid = lax.axis_index("s") * NC + lax.axis_index("c")
    base = wid * PER_W

    pltpu.sync_copy(x_hbm.at[wid], idx_all)

    def gather(j, rows, sem):
        return pltpu.make_async_copy(table_hbm.at[idx_all.at[j]], rows, sem)

    def scatter(j, rows, sem):
        return pltpu.make_async_copy(
            rows, out_hbm.at[pl.ds(base + j * CHUNK, CHUNK)], sem)

    gather(0, rows_g0, sg0).start()
    gather(1, rows_g1, sg1).start()

    def group(g, carry):
        for b, rg, rs, sg, ss in ((0, rows_g0, rows_s0, sg0, ss0),
                                  (1, rows_g1, rows_s1, sg1, ss1)):
            i = g * 2 + b
            gather(i, rg, sg).wait()

            @pl.when(g >= 1)
            def _():
                scatter(i - 2, rs, ss).wait()

            def scale_row(r, c):
                for j in range(D // LANES):
                    sl = pl.ds(j * LANES, LANES)
                    rs[r, sl] = rg[r, sl] * SCALE
                return c

            lax.fori_loop(0, CHUNK, scale_row, 0, unroll=2)
            scatter(i, rs, ss).start()

            @pl.when(g <= NGRP - 2)
            def _():
                gather(i + 2, rg, sg).start()
        return carry

    lax.fori_loop(0, NGRP, group, 0)

    scatter(NCHUNK - 2, rows_s0, ss0).wait()
    scatter(NCHUNK - 1, rows_s1, ss1).wait()


@jax.jit
def _emb(x_resh, table):
    mesh = plsc.VectorSubcoreMesh(core_axis_name="c", subcore_axis_name="s")
    run = functools.partial(
        pl.kernel,
        mesh=mesh,
        out_type=jax.ShapeDtypeStruct((TOTAL, D), jnp.float32),
        scratch_types=[
            pltpu.VMEM((NCHUNK, CHUNK), jnp.int32),
            pltpu.VMEM((CHUNK, D), jnp.float32),
            pltpu.VMEM((CHUNK, D), jnp.float32),
            pltpu.VMEM((CHUNK, D), jnp.float32),
            pltpu.VMEM((CHUNK, D), jnp.float32),
            pltpu.SemaphoreType.DMA,
            pltpu.SemaphoreType.DMA,
            pltpu.SemaphoreType.DMA,
            pltpu.SemaphoreType.DMA,
        ],
    )(_emb_body)
    return run(x_resh, table)


def kernel(X, table):
    out = _emb(X.reshape(NW, NCHUNK, CHUNK), table)
    return out.reshape(B, H, D)


# speedup vs baseline: 1.6554x; 1.6554x over previous
import functools

import jax
import jax.numpy as jnp
from jax import lax
from jax.experimental import pallas as pl
from jax.experimental.pallas import tpu as pltpu
from jax.experimental.pallas import tpu_sc as plsc

N_ITEMS = 100001
D = 128
B = 4096
H = 200
TOTAL = B * H
SCALE = 50.0 ** 0.5

NC = 2
NS = 16
NW = NC * NS
PER_W = TOTAL // NW
CHUNK = 128
NCHUNK = PER_W // CHUNK
LANES = 16


def _emb_body(x_hbm, table_hbm, out_hbm, idx_all, rows_v, sem):
    wid = lax.axis_index("s") * NC + lax.axis_index("c")
    base = wid * PER_W

    pltpu.sync_copy(x_hbm.at[wid], idx_all)

    def step(i, carry):
        off = base + i * CHUNK
        pltpu.async_copy(table_hbm.at[idx_all.at[i]], rows_v, sem).wait()

        def scale_row(r, c):
            for j in range(D // LANES):
                sl = pl.ds(j * LANES, LANES)
                rows_v[r, sl] = rows_v[r, sl] * SCALE
            return c

        lax.fori_loop(0, CHUNK, scale_row, 0, unroll=2)
        pltpu.sync_copy(rows_v, out_hbm.at[pl.ds(off, CHUNK)])
        return carry

    lax.fori_loop(0, NCHUNK, step, 0)


@jax.jit
def _emb(x_resh, table):
    mesh = plsc.VectorSubcoreMesh(core_axis_name="c", subcore_axis_name="s")
    run = functools.partial(
        pl.kernel,
        mesh=mesh,
        out_type=jax.ShapeDtypeStruct((TOTAL, D), jnp.float32),
        scratch_types=[
            pltpu.VMEM((NCHUNK, CHUNK), jnp.int32),
            pltpu.VMEM((CHUNK, D), jnp.float32),
            pltpu.SemaphoreType.DMA,
        ],
    )(_emb_body)
    return run(x_resh, table)


def kernel(X, table):
    out = _emb(X.reshape(NW, NCHUNK, CHUNK), table)
    return out.reshape(B, H, D)


# speedup vs baseline: 3.0257x; 1.8278x over previous
import functools

import jax
import jax.numpy as jnp
from jax import lax
from jax.experimental import pallas as pl
from jax.experimental.pallas import tpu as pltpu
from jax.experimental.pallas import tpu_sc as plsc

N_ITEMS = 100001
D = 128
B = 4096
H = 200
TOTAL = B * H
SCALE = 50.0 ** 0.5

NC = 2
NS = 16
NW = NC * NS
PER_W = TOTAL // NW
CHUNK = 128
NCHUNK = PER_W // CHUNK
NRING = 4
LANES = 16


def _emb_body(x_hbm, table_hbm, out_hbm, idx_all, rows_v, sem_g, sem_s):
    wid = lax.axis_index("s") * NC + lax.axis_index("c")
    base = wid * PER_W

    pltpu.sync_copy(x_hbm.at[wid], idx_all)

    def gather(j, slot):
        return pltpu.make_async_copy(
            table_hbm.at[idx_all.at[j]],
            rows_v.at[pl.ds(slot * CHUNK, CHUNK)], sem_g)

    def scatter(j, slot):
        return pltpu.make_async_copy(
            rows_v.at[pl.ds(slot * CHUNK, CHUNK)],
            out_hbm.at[pl.ds(base + j * CHUNK, CHUNK)], sem_s)

    gather(0, 0).start()
    gather(1, 1).start()

    def step(i, carry):
        h = lax.rem(i, NRING)
        gather(i, h).wait()
        rbase = h * CHUNK

        def scale_row(r, c):
            for j in range(D // LANES):
                sl = pl.ds(j * LANES, LANES)
                rows_v[rbase + r, sl] = rows_v[rbase + r, sl] * SCALE
            return c

        lax.fori_loop(0, CHUNK, scale_row, 0, unroll=2)
        scatter(i, h).start()

        @pl.when(i >= 2)
        def _():
            scatter(i - 2, lax.rem(i - 2, NRING)).wait()

        @pl.when(i + 2 < NCHUNK)
        def _():
            gather(i + 2, lax.rem(i + 2, NRING)).start()

        return carry

    lax.fori_loop(0, NCHUNK, step, 0)

    scatter(NCHUNK - 2, lax.rem(NCHUNK - 2, NRING)).wait()
    scatter(NCHUNK - 1, lax.rem(NCHUNK - 1, NRING)).wait()


@jax.jit
def _emb(x_resh, table):
    mesh = plsc.VectorSubcoreMesh(core_axis_name="c", subcore_axis_name="s")
    run = functools.partial(
        pl.kernel,
        mesh=mesh,
        out_type=jax.ShapeDtypeStruct((TOTAL, D), jnp.float32),
        scratch_types=[
            pltpu.VMEM((NCHUNK, CHUNK), jnp.int32),
            pltpu.VMEM((NRING * CHUNK, D), jnp.float32),
            pltpu.SemaphoreType.DMA,
            pltpu.SemaphoreType.DMA,
        ],
    )(_emb_body)
    return run(x_resh, table)


def kernel(X, table):
    out = _emb(X.reshape(NW, NCHUNK, CHUNK), table)
    return out.reshape(B, H, D)


# speedup vs baseline: 3.0356x; 1.0033x over previous
import functools

import jax
import jax.numpy as jnp
from jax import lax
from jax.experimental import pallas as pl
from jax.experimental.pallas import tpu as pltpu
from jax.experimental.pallas import tpu_sc as plsc

N_ITEMS = 100001
D = 128
B = 4096
H = 200
TOTAL = B * H
SCALE = 50.0 ** 0.5

NC = 2
NS = 16
NW = NC * NS
PER_W = TOTAL // NW
CHUNK = 128
NCHUNK = PER_W // CHUNK
NRING = 6
LANES = 16


def _emb_body(x_hbm, table_hbm, out_hbm, idx_all, rows_v, sem_g, sem_s):
    wid = lax.axis_index("s") * NC + lax.axis_index("c")
    base = wid * PER_W

    pltpu.sync_copy(x_hbm.at[wid], idx_all)

    def gather(j, slot):
        return pltpu.make_async_copy(
            table_hbm.at[idx_all.at[j]],
            rows_v.at[pl.ds(slot * CHUNK, CHUNK)], sem_g)

    def scatter(j, slot):
        return pltpu.make_async_copy(
            rows_v.at[pl.ds(slot * CHUNK, CHUNK)],
            out_hbm.at[pl.ds(base + j * CHUNK, CHUNK)], sem_s)

    gather(0, 0).start()
    gather(1, 1).start()
    gather(2, 2).start()

    def step(i, carry):
        h = lax.rem(i, NRING)
        gather(i, h).wait()

        @pl.when(i >= 3)
        def _():
            scatter(i - 3, lax.rem(i - 3, NRING)).wait()

        @pl.when(i + 3 < NCHUNK)
        def _():
            gather(i + 3, lax.rem(i + 3, NRING)).start()

        rbase = h * CHUNK

        def scale_row(r, c):
            for j in range(D // LANES):
                sl = pl.ds(j * LANES, LANES)
                rows_v[rbase + r, sl] = rows_v[rbase + r, sl] * SCALE
            return c

        lax.fori_loop(0, CHUNK, scale_row, 0, unroll=4)
        scatter(i, h).start()
        return carry

    lax.fori_loop(0, NCHUNK, step, 0)

    scatter(NCHUNK - 3, lax.rem(NCHUNK - 3, NRING)).wait()
    scatter(NCHUNK - 2, lax.rem(NCHUNK - 2, NRING)).wait()
    scatter(NCHUNK - 1, lax.rem(NCHUNK - 1, NRING)).wait()


@jax.jit
def _emb(x_resh, table):
    mesh = plsc.VectorSubcoreMesh(core_axis_name="c", subcore_axis_name="s")
    run = functools.partial(
        pl.kernel,
        mesh=mesh,
        out_type=jax.ShapeDtypeStruct((TOTAL, D), jnp.float32),
        scratch_types=[
            pltpu.VMEM((NCHUNK, CHUNK), jnp.int32),
            pltpu.VMEM((NRING * CHUNK, D), jnp.float32),
            pltpu.SemaphoreType.DMA,
            pltpu.SemaphoreType.DMA,
        ],
    )(_emb_body)
    return run(x_resh, table)


def kernel(X, table):
    out = _emb(X.reshape(NW, NCHUNK, CHUNK), table)
    return out.reshape(B, H, D)
